# baseline (speedup 1.0000x reference)
import functools

import jax
import jax.numpy as jnp
from jax import lax
from jax.experimental import pallas as pl
from jax.experimental.pallas import tpu as pltpu
from jax.experimental.pallas import tpu_sc as plsc


def kernel(subset, embedding):
    (B,) = subset.shape
    V, D = embedding.shape
    L = 16

    view = embedding.reshape(V // 2, 2 * D)

    info = plsc.get_sparse_core_info()
    NC, NS = info.num_cores, info.num_subcores
    NW = NC * NS
    b_per_w = B // NW
    C = 4
    ch = b_per_w // C
    gq = ch // L

    mesh = plsc.VectorSubcoreMesh(core_axis_name="c", subcore_axis_name="s")

    @functools.partial(
        pl.kernel,
        mesh=mesh,
        out_type=jax.ShapeDtypeStruct((B, D), jnp.float32),
        scratch_types=[
            pltpu.VMEM((b_per_w,), jnp.int32),
            pltpu.VMEM((C, ch), jnp.int32),
            pltpu.VMEM((b_per_w,), jnp.int32),
            pltpu.VMEM((ch, 2 * D), jnp.float32),
            pltpu.VMEM((ch, 2 * D), jnp.float32),
            pltpu.VMEM((ch, D), jnp.float32),
            pltpu.SemaphoreType.DMA,
            pltpu.SemaphoreType.DMA,
        ],
    )
    def gather_kernel(idx_hbm, view_hbm, out_hbm, idx_v, pair_v, off_v,
                      rows_a, rows_b, out_v, sem_a, sem_b):
        wid = lax.axis_index("s") * NC + lax.axis_index("c")
        base = wid * b_per_w
        pltpu.sync_copy(idx_hbm.at[pl.ds(base, b_per_w)], idx_v)

        def prep(q, carry):
            x = idx_v[pl.ds(q * L, L)]
            off_v[pl.ds(q * L, L)] = lax.shift_left(lax.bitwise_and(x, 1), 6)
            return carry

        for c in range(C):
            def prep_pair(q, carry, c=c):
                x = idx_v[pl.ds(c * ch + q * L, L)]
                pair_v[c, pl.ds(q * L, L)] = lax.shift_right_logical(x, 1)
                return carry
            lax.fori_loop(0, gq, prep_pair, 0)
        lax.fori_loop(0, C * gq, prep, 0)

        bufs = [rows_a, rows_b]
        sems = [sem_a, sem_b]

        def start(c):
            return pltpu.async_copy(view_hbm.at[pair_v.at[c]],
                                    bufs[c % 2], sems[c % 2])

        def select_and_store(c, rows_v):
            def select(g, carry):
                off_vec = off_v[pl.ds(c * ch + g * L, L)]
                for l in range(L):
                    b = g * L + l
                    off = off_vec[l]
                    for j in range(D // L):
                        out_v[b, pl.ds(j * L, L)] = (
                            rows_v[b, pl.ds(off + j * L, L)])
                return carry

            lax.fori_loop(0, gq, select, 0)
            pltpu.sync_copy(out_v, out_hbm.at[pl.ds(base + c * ch, ch)])

        copies = {}
        copies[0] = start(0)
        copies[1] = start(1)
        for c in range(C):
            copies[c].wait()
            select_and_store(c, bufs[c % 2])
            if c + 2 < C:
                copies[c + 2] = start(c + 2)

    return gather_kernel(subset.astype(jnp.int32), view)


# speedup vs baseline: 2.2437x; 2.2437x over previous
import functools

import jax
import jax.numpy as jnp
from jax import lax
from jax.experimental import pallas as pl
from jax.experimental.pallas import tpu as pltpu
from jax.experimental.pallas import tpu_sc as plsc


def kernel(subset, embedding):
    (B,) = subset.shape
    V, D = embedding.shape
    L = 16
    BLK = 128
    NBLK = (V + BLK - 1) // BLK
    RING = 128

    tabT = embedding.T

    info = plsc.get_sparse_core_info()
    NC, NS = info.num_cores, info.num_subcores
    NW = NC * NS

    mesh = plsc.VectorSubcoreMesh(core_axis_name="c", subcore_axis_name="s")

    @functools.partial(
        pl.kernel,
        mesh=mesh,
        out_type=jax.ShapeDtypeStruct((B, D), jnp.float32),
        compiler_params=pltpu.CompilerParams(needs_layout_passes=False),
        scratch_types=[
            pltpu.VMEM((B,), jnp.int32),
            pltpu.VMEM((B + L,), jnp.int32),
            pltpu.VMEM((B + L,), jnp.int32),
            pltpu.VMEM((D, BLK), jnp.float32),
            pltpu.VMEM((D, BLK), jnp.float32),
            pltpu.VMEM((RING, D), jnp.float32),
            pltpu.VMEM((D, V % BLK), jnp.float32),
            pltpu.VMEM((2 * L,), jnp.int32),
            pltpu.VMEM((2 * L,), jnp.int32),
            pltpu.SemaphoreType.DMA,
            pltpu.SemaphoreType.DMA,
            pltpu.SemaphoreType.DMA,
        ],
    )
    def scan_kernel(idx_hbm, tab_hbm, out_hbm, idx_all, hit_i, hit_b,
                    slab_a, slab_b, ring, tail_slab, tmpc, tmpd,
                    sem_a, sem_b, sem_o):
        wid = lax.axis_index("s") * NC + lax.axis_index("c")
        lo = (wid * NBLK) // NW
        hi = ((wid + 1) * NBLK) // NW
        hi_main = jnp.minimum(hi, NBLK - 1)
        nmain = hi_main - lo
        lane = lax.iota(jnp.int32, L)

        pltpu.sync_copy(idx_hbm, idx_all)

        def start(mb, slab, sem):
            off = pl.multiple_of(mb * BLK, BLK)
            return pltpu.async_copy(tab_hbm.at[:, pl.ds(off, BLK)], slab, sem)

        def drain_block(slab, sem):
            pltpu.make_async_copy(tab_hbm.at[:, pl.ds(0, BLK)], slab,
                                  sem).wait()

        @pl.when(nmain > 0)
        def _():
            start(lo, slab_a, sem_a)

        @pl.when(nmain > 1)
        def _():
            start(lo + 1, slab_b, sem_b)

        def detect(q, cur):
            v = idx_all[pl.ds(q * L, L)]
            m = (v >= lo * BLK) & (v < hi * BLK)
            n = plsc.all_reduce_population_count(m)[0]
            plsc.store_compressed(hit_i.at[pl.ds(cur, L)], v, mask=m)
            plsc.store_compressed(hit_b.at[pl.ds(cur, L)], q * L + lane,
                                  mask=m)
            return cur + n

        nh = lax.fori_loop(0, B // L, detect, jnp.int32(0))
        nchunks = (nh + L - 1) // L

        def process(mb, slab, w):
            def pchunk(p, w):
                hv = hit_i[pl.ds(p * L, L)]
                hb = hit_b[pl.ds(p * L, L)]
                m = (lax.shift_right_logical(hv, 7) == mb) & (
                    (p * L + lane) < nh)
                k2 = plsc.all_reduce_population_count(m)[0]
                plsc.store_compressed(tmpc.at[pl.ds(0, L)],
                                      lax.bitwise_and(hv, BLK - 1), mask=m)
                plsc.store_compressed(tmpd.at[pl.ds(0, L)], hb, mask=m)

                def hloop(h, w):
                    c = tmpc[pl.ds(h, L)][0]
                    d = tmpd[pl.ds(h, L)][0]
                    slot = lax.bitwise_and(w, RING - 1)

                    @pl.when(w >= RING)
                    def _():
                        pltpu.make_async_copy(tab_hbm.at[0, pl.ds(0, D)],
                                              ring.at[0], sem_o).wait()

                    cvec = jnp.full((L,), 0, jnp.int32) + c
                    for jq in range(D // L):
                        vals = plsc.load_gather(slab, [jq * L + lane, cvec])
                        ring[slot, pl.ds(jq * L, L)] = vals
                    pltpu.async_copy(ring.at[slot], out_hbm.at[d], sem_o)
                    return w + 1

                return lax.fori_loop(0, k2, hloop, w)

            return lax.fori_loop(0, nchunks, pchunk, w)

        def mainloop(t, w):
            mb0 = lo + 2 * t
            drain_block(slab_a, sem_a)
            w = process(mb0, slab_a, w)

            @pl.when(2 * t + 2 < nmain)
            def _():
                start(mb0 + 2, slab_a, sem_a)

            drain_block(slab_b, sem_b)
            w = process(mb0 + 1, slab_b, w)

            @pl.when(2 * t + 3 < nmain)
            def _():
                start(mb0 + 3, slab_b, sem_b)

            return w

        w = lax.fori_loop(0, nmain // 2, mainloop, jnp.int32(0))

        def oddloop(r, w):
            drain_block(slab_a, sem_a)
            return process(hi_main - 1, slab_a, w)

        w = lax.fori_loop(0, lax.bitwise_and(nmain, 1), oddloop, w)

        def tailloop(r, w):
            pltpu.async_copy(
                tab_hbm.at[:, pl.ds((NBLK - 1) * BLK, V - (NBLK - 1) * BLK)],
                tail_slab, sem_a).wait()
            return process(jnp.int32(NBLK - 1), tail_slab, w)

        w = lax.fori_loop(0, jnp.int32(1) * (hi == NBLK), tailloop, w)

        nout = jnp.minimum(w, RING)
        for i in range(RING):
            @pl.when(i < nout)
            def _():
                pltpu.make_async_copy(tab_hbm.at[0, pl.ds(0, D)],
                                      ring.at[0], sem_o).wait()

    return scan_kernel(subset.astype(jnp.int32), tabT)


# speedup vs baseline: 2.3366x; 1.0414x over previous
import functools

import jax
import jax.numpy as jnp
from jax import lax
from jax.experimental import pallas as pl
from jax.experimental.pallas import tpu as pltpu
from jax.experimental.pallas import tpu_sc as plsc


def kernel(subset, embedding):
    (B,) = subset.shape
    V, D = embedding.shape
    L = 16
    BLK = 128
    NBLK = (V + BLK - 1) // BLK
    RING = 64
    K = 6

    tabT = embedding.T

    info = plsc.get_sparse_core_info()
    NC, NS = info.num_cores, info.num_subcores
    NW = NC * NS

    mesh = plsc.VectorSubcoreMesh(core_axis_name="c", subcore_axis_name="s")

    slab_types = [pltpu.VMEM((D, BLK), jnp.float32) for _ in range(K)]
    sem_types = [pltpu.SemaphoreType.DMA for _ in range(K)]

    @functools.partial(
        pl.kernel,
        mesh=mesh,
        out_type=jax.ShapeDtypeStruct((B, D), jnp.float32),
        compiler_params=pltpu.CompilerParams(needs_layout_passes=False),
        scratch_types=[
            pltpu.VMEM((B,), jnp.int32),
            pltpu.VMEM((B + L,), jnp.int32),
            pltpu.VMEM((B + L,), jnp.int32),
            *slab_types,
            pltpu.VMEM((RING, D), jnp.float32),
            pltpu.VMEM((D, V % BLK), jnp.float32),
            pltpu.VMEM((2 * L,), jnp.int32),
            pltpu.VMEM((2 * L,), jnp.int32),
            *sem_types,
            pltpu.SemaphoreType.DMA,
        ],
    )
    def scan_kernel(idx_hbm, tab_hbm, out_hbm, idx_all, hit_i, hit_b,
                    *rest):
        slabs = rest[:K]
        ring, tail_slab, tmpc, tmpd = rest[K:K + 4]
        sems = rest[K + 4:2 * K + 4]
        sem_o = rest[2 * K + 4]

        wid = lax.axis_index("s") * NC + lax.axis_index("c")
        lo = (wid * NBLK) // NW
        hi = ((wid + 1) * NBLK) // NW
        hi_main = jnp.minimum(hi, NBLK - 1)
        nmain = hi_main - lo
        lane = lax.iota(jnp.int32, L)

        pltpu.sync_copy(idx_hbm, idx_all)

        def start(mb, slab, sem):
            off = pl.multiple_of(mb * BLK, BLK)
            return pltpu.async_copy(tab_hbm.at[:, pl.ds(off, BLK)], slab, sem)

        def drain_block(slab, sem):
            pltpu.make_async_copy(tab_hbm.at[:, pl.ds(0, BLK)], slab,
                                  sem).wait()

        for k in range(K):
            @pl.when(k < nmain)
            def _(k=k):
                start(lo + k, slabs[k], sems[k])

        def detect(q, cur):
            v = idx_all[pl.ds(q * L, L)]
            m = (v >= lo * BLK) & (v < hi * BLK)
            n = plsc.all_reduce_population_count(m)[0]
            plsc.store_compressed(hit_i.at[pl.ds(cur, L)], v, mask=m)
            plsc.store_compressed(hit_b.at[pl.ds(cur, L)], q * L + lane,
                                  mask=m)
            return cur + n

        nh = lax.fori_loop(0, B // L, detect, jnp.int32(0))
        hit_i[pl.ds(nh, L)] = jnp.full((L,), jnp.int32(0x7FFFFFFF))
        nchunks = (nh + L - 1) // L

        def process(mb, slab, w):
            def pchunk(p, w):
                hv = hit_i[pl.ds(p * L, L)]
                m = lax.shift_right_logical(hv, 7) == mb
                k2 = plsc.all_reduce_population_count(m)[0]

                def hits_found(r, w):
                    hb = hit_b[pl.ds(p * L, L)]
                    plsc.store_compressed(tmpc.at[pl.ds(0, L)],
                                          lax.bitwise_and(hv, BLK - 1),
                                          mask=m)
                    plsc.store_compressed(tmpd.at[pl.ds(0, L)], hb, mask=m)

                    def hloop(h, w):
                        c = tmpc[pl.ds(h, L)][0]
                        d = tmpd[pl.ds(h, L)][0]
                        slot = lax.bitwise_and(w, RING - 1)

                        @pl.when(w >= RING)
                        def _():
                            pltpu.make_async_copy(tab_hbm.at[0, pl.ds(0, D)],
                                                  ring.at[0], sem_o).wait()

                        cvec = jnp.full((L,), 0, jnp.int32) + c
                        for jq in range(D // L):
                            vals = plsc.load_gather(slab,
                                                    [jq * L + lane, cvec])
                            ring[slot, pl.ds(jq * L, L)] = vals
                        pltpu.async_copy(ring.at[slot], out_hbm.at[d], sem_o)
                        return w + 1

                    return lax.fori_loop(0, k2, hloop, w)

                return lax.fori_loop(0, (k2 > 0).astype(jnp.int32),
                                     hits_found, w)

            return lax.fori_loop(0, nchunks, pchunk, w)

        def mainloop(t, w):
            for k in range(K):
                pos = K * t + k
                mb = lo + pos

                def body(r, w, k=k, mb=mb):
                    drain_block(slabs[k], sems[k])
                    return process(mb, slabs[k], w)

                w = lax.fori_loop(0, (pos < nmain).astype(jnp.int32), body, w)

                @pl.when(pos + K < nmain)
                def _(k=k, mb=mb):
                    start(mb + K, slabs[k], sems[k])
            return w

        w = lax.fori_loop(0, (nmain + K - 1) // K, mainloop, jnp.int32(0))

        def tailloop(r, w):
            pltpu.async_copy(
                tab_hbm.at[:, pl.ds((NBLK - 1) * BLK, V - (NBLK - 1) * BLK)],
                tail_slab, sems[0]).wait()
            return process(jnp.int32(NBLK - 1), tail_slab, w)

        w = lax.fori_loop(0, (hi == NBLK).astype(jnp.int32), tailloop, w)

        nout = jnp.minimum(w, RING)
        for i in range(RING):
            @pl.when(i < nout)
            def _():
                pltpu.make_async_copy(tab_hbm.at[0, pl.ds(0, D)],
                                      ring.at[0], sem_o).wait()

    return scan_kernel(subset.astype(jnp.int32), tabT)


# speedup vs baseline: 4.5912x; 1.9649x over previous
import functools

import jax
import jax.numpy as jnp
from jax import lax
from jax.experimental import pallas as pl
from jax.experimental.pallas import tpu as pltpu
from jax.experimental.pallas import tpu_sc as plsc


def kernel(subset, embedding):
    (B,) = subset.shape
    V, D = embedding.shape
    L = 16
    BLK = 128
    NBLK = (V + BLK - 1) // BLK
    RING = 32
    K = 5
    U = 4

    tabT = embedding.T

    info = plsc.get_sparse_core_info()
    NC, NS = info.num_cores, info.num_subcores
    NW = NC * NS

    mesh = plsc.VectorSubcoreMesh(core_axis_name="c", subcore_axis_name="s")

    @functools.partial(
        pl.kernel,
        mesh=mesh,
        out_type=jax.ShapeDtypeStruct((B, D), jnp.float32),
        compiler_params=pltpu.CompilerParams(needs_layout_passes=False),
        scratch_types=[
            pltpu.VMEM((B,), jnp.int32),
            pltpu.VMEM((B + U * L,), jnp.int32),
            pltpu.VMEM((K, D, BLK), jnp.float32),
            pltpu.VMEM((K, D, BLK), jnp.float32),
            pltpu.VMEM((RING, D), jnp.float32),
            pltpu.VMEM((D, V % BLK), jnp.float32),
            pltpu.VMEM((2 * L,), jnp.int32),
            pltpu.SemaphoreType.DMA,
            pltpu.SemaphoreType.DMA,
            pltpu.SemaphoreType.DMA,
        ],
    )
    def scan_kernel(idx_hbm, tab_hbm, out_hbm, idx_all, hit_p, slab_a,
                    slab_b, ring, tail_slab, tmpc, sem_a, sem_b, sem_o):
        wid = lax.axis_index("s") * NC + lax.axis_index("c")
        lo = (wid * NBLK) // NW
        hi = ((wid + 1) * NBLK) // NW
        hi_main = jnp.minimum(hi, NBLK - 1)
        nmain = hi_main - lo
        lane = lax.iota(jnp.int32, L)

        pltpu.sync_copy(idx_hbm, idx_all)

        def start(pos, slab3, k, sem):
            mb = lo + pos
            off = pl.multiple_of(mb * BLK, BLK)
            pltpu.async_copy(tab_hbm.at[:, pl.ds(off, BLK)], slab3.at[k], sem)

        def drain_block(slab3, k, sem):
            pltpu.make_async_copy(tab_hbm.at[:, pl.ds(0, BLK)], slab3.at[k],
                                  sem).wait()

        def drain_out_one():
            pltpu.make_async_copy(tab_hbm.at[0, pl.ds(0, D)], ring.at[0],
                                  sem_o).wait()

        for k in range(K):
            @pl.when(k < nmain)
            def _(k=k):
                start(k, slab_a, k, sem_a)
        for k in range(K):
            @pl.when(K + k < nmain)
            def _(k=k):
                start(K + k, slab_b, k, sem_b)

        def detect(q, cur):
            v = idx_all[pl.ds(q * L, L)]
            m = (v >= lo * BLK) & (v < hi * BLK)
            n = plsc.all_reduce_population_count(m)[0]
            packed = lax.shift_left(v - lo * BLK, 14) + (q * L + lane)
            plsc.store_compressed(hit_p.at[pl.ds(cur, L)], packed, mask=m)
            return cur + n

        nh = lax.fori_loop(0, B // L, detect, jnp.int32(0))
        sent = jnp.full((L,), jnp.int32(0x7FFFFFFF))
        for u in range(U):
            hit_p[pl.ds(nh + u * L, L)] = sent
        nchunks = (nh + U * L - 1) // (U * L)

        def hit_row(v, gather):
            def run(w):
                c = lax.bitwise_and(lax.shift_right_logical(v, 14), BLK - 1)
                d = lax.bitwise_and(v, B - 1)
                slot = lax.bitwise_and(w, RING - 1)

                @pl.when(w >= RING)
                def _():
                    drain_out_one()

                for jq in range(D // L):
                    vals = gather(jq * L + lane, c)
                    ring[slot, pl.ds(jq * L, L)] = vals
                pltpu.async_copy(ring.at[slot], out_hbm.at[d], sem_o)
                return w + 1

            return run

        def process_group(g, slab3, w):
            b0 = g * K
            bhi = jnp.minimum(b0 + K, nmain)

            def pchunk(p, w):
                for u in range(U):
                    hv = hit_p[pl.ds((p * U + u) * L, L)]
                    grel = lax.shift_right_logical(hv, 21)
                    m = (grel >= b0) & (grel < bhi)
                    k2 = plsc.all_reduce_population_count(m)[0]

                    def found(r, w, hv=hv, m=m):
                        plsc.store_compressed(tmpc.at[pl.ds(0, L)], hv,
                                              mask=m)

                        def hloop(h, w):
                            v = tmpc[pl.ds(h, L)][0]
                            krel = lax.shift_right_logical(v, 21) - b0

                            def gather(rows, c):
                                return plsc.load_gather(
                                    slab3,
                                    [jnp.full((L,), 0, jnp.int32) + krel,
                                     rows,
                                     jnp.full((L,), 0, jnp.int32) + c])

                            return hit_row(v, gather)(w)

                        return lax.fori_loop(0, k2, hloop, w)

                    w = lax.fori_loop(0, (k2 > 0).astype(jnp.int32), found, w)
                return w

            return lax.fori_loop(0, nchunks, pchunk, w)

        def mainloop(t, w):
            for parity, (slab3, sem) in enumerate([(slab_a, sem_a),
                                                   (slab_b, sem_b)]):
                g = 2 * t + parity
                for k in range(K):
                    @pl.when(g * K + k < nmain)
                    def _(k=k):
                        drain_block(slab3, k, sem)

                def body(r, w, g=g, slab3=slab3):
                    return process_group(g, slab3, w)

                w = lax.fori_loop(0, (g * K < nmain).astype(jnp.int32),
                                  body, w)
                for k in range(K):
                    @pl.when((g + 2) * K + k < nmain)
                    def _(k=k, g=g):
                        start((g + 2) * K + k, slab3, k, sem)
            return w

        ngroups = (nmain + K - 1) // K
        w = lax.fori_loop(0, (ngroups + 1) // 2, mainloop, jnp.int32(0))

        def tailloop(r, w):
            pltpu.async_copy(
                tab_hbm.at[:, pl.ds((NBLK - 1) * BLK, V - (NBLK - 1) * BLK)],
                tail_slab, sem_a).wait()

            def tchunk(p, w):
                hv = hit_p[pl.ds(p * L, L)]
                m = lax.shift_right_logical(hv, 21) == nmain
                k2 = plsc.all_reduce_population_count(m)[0]

                def found(r, w, hv=hv, m=m):
                    plsc.store_compressed(tmpc.at[pl.ds(0, L)], hv, mask=m)

                    def hloop(h, w):
                        v = tmpc[pl.ds(h, L)][0]

                        def gather(rows, c):
                            return plsc.load_gather(
                                tail_slab,
                                [rows, jnp.full((L,), 0, jnp.int32) + c])

                        return hit_row(v, gather)(w)

                    return lax.fori_loop(0, k2, hloop, w)

                return lax.fori_loop(0, (k2 > 0).astype(jnp.int32), found, w)

            return lax.fori_loop(0, (nh + L - 1) // L, tchunk, w)

        w = lax.fori_loop(0, (hi == NBLK).astype(jnp.int32), tailloop, w)

        nout = jnp.minimum(w, RING)
        for i in range(RING):
            @pl.when(i < nout)
            def _():
                drain_out_one()

    return scan_kernel(subset.astype(jnp.int32), tabT)
